# baseline (speedup 1.0000x reference)
import jax
import jax.numpy as jnp
from jax import lax
from jax.experimental import pallas as pl
from jax.experimental.pallas import tpu as pltpu
from jax.experimental.pallas import tpu_sc as plsc

FROZEN = 900000
UNFROZEN = 100000
DIM = 64
BATCH = 16384
HIST = 50
B_ROWS = BATCH * HIST

NC, NS = 2, 16
NW = NC * NS
PW = B_ROWS // NW
S = 512
NCH = PW // S
G = S // 16
NJ = S // 128
UROWS = (PW + 128 + 127) // 128


def _body(idx_h, wf_h, wu_h, out_h,
          idx_v, fidx_v, rows_v, upos_v, rawi_v, uix_v, urows_v,
          sem_i, sem_g0, sem_g1, sem_w0, sem_w1,
          sem_p0, sem_p1, sem_r0, sem_r1, sem_s0, sem_s1):
    sem_g = (sem_g0, sem_g1)
    sem_w = (sem_w0, sem_w1)
    sem_p = (sem_p0, sem_p1)
    sem_r = (sem_r0, sem_r1)
    sem_s = (sem_s0, sem_s1)
    cid = lax.axis_index("c")
    sid = lax.axis_index("s")
    wid = sid * NC + cid
    wbase = wid * PW
    iota16 = lax.iota(jnp.int32, 16)

    def compact(c, buf, u_off):
        base = wbase + c * S
        for g in range(G):
            v = idx_v[buf, pl.ds(g * 16, 16)]
            mu = v >= FROZEN
            fidx_v[buf, g // 8, pl.ds((g % 8) * 16, 16)] = jnp.minimum(v, FROZEN - 1)
            pos = base + g * 16 + iota16
            cs = plsc.cumsum(jnp.where(mu, jnp.int32(1), jnp.int32(0)))
            dst = u_off + cs - 1
            dr = lax.shift_right_logical(dst, 7)
            dc = lax.bitwise_and(dst, 127)
            plsc.store_scatter(upos_v, [dr, dc], pos, mask=mu)
            u_off = u_off + cs[15]
        return u_off

    def fire_gathers(b):
        for j in range(NJ):
            pltpu.async_copy(wf_h.at[fidx_v.at[b, j]],
                             rows_v.at[b, pl.ds(j * 128, 128)], sem_g[b])

    def wait_gathers(b):
        for j in range(NJ):
            pltpu.make_async_copy(wf_h.at[fidx_v.at[b, j]],
                                  rows_v.at[b, pl.ds(j * 128, 128)],
                                  sem_g[b]).wait()

    def fire_write(c, b):
        pltpu.async_copy(rows_v.at[b], out_h.at[pl.ds(wbase + c * S, S)],
                         sem_w[b])

    def wait_write(b):
        pltpu.make_async_copy(rows_v.at[b], out_h.at[pl.ds(0, S)],
                              sem_w[b]).wait()

    def fire_idx_load(c, b):
        pltpu.async_copy(idx_h.at[pl.ds(wbase + c * S, S)], idx_v.at[b], sem_i)

    def wait_idx_load(b):
        pltpu.make_async_copy(idx_h.at[pl.ds(0, S)], idx_v.at[b], sem_i).wait()

    pltpu.sync_copy(idx_h.at[pl.ds(wbase, S)], idx_v.at[0])
    u_off = compact(0, 0, jnp.int32(0))
    fire_gathers(0)
    fire_idx_load(1, 1)

    def pair_body(p, u_off):
        for b in (0, 1):
            c = 2 * p + b
            nb = 1 - b
            wait_idx_load(nb)
            u_off = compact(c + 1, nb, u_off)
            fire_idx_load(c + 2, b)

            @pl.when(c >= 1)
            def _():
                wait_write(nb)

            fire_gathers(nb)
            wait_gathers(b)
            fire_write(c, b)
        return u_off

    u_off = lax.fori_loop(0, (NCH - 2) // 2, pair_body, u_off)

    wait_idx_load(1)
    n_u = compact(NCH - 1, 1, u_off)
    wait_write(1)
    fire_gathers(1)
    wait_gathers(0)
    fire_write(NCH - 2, 0)
    wait_gathers(1)
    fire_write(NCH - 1, 1)
    wait_write(0)
    wait_write(1)

    @pl.when(n_u > 0)
    def _pass2():
        nm1 = n_u - 1
        lr = jnp.full((16,), lax.shift_right_logical(nm1, 7), jnp.int32)
        lc = jnp.full((16,), lax.bitwise_and(nm1, 127), jnp.int32)
        vp = plsc.load_gather(upos_v, [lr, lc])
        for j in range(8):
            dd = n_u + j * 16 + iota16
            ddr = lax.shift_right_logical(dd, 7)
            ddc = lax.bitwise_and(dd, 127)
            plsc.store_scatter(upos_v, [ddr, ddc], vp)
        n_ch2 = (n_u + 127) // 128

        def fire_pos_gather(cu, b):
            pltpu.async_copy(idx_h.at[upos_v.at[cu]], rawi_v.at[b], sem_p[b])

        def wait_pos_gather(b):
            pltpu.make_async_copy(idx_h.at[upos_v.at[0]], rawi_v.at[b],
                                  sem_p[b]).wait()

        fire_pos_gather(0, 0)

        def p2pair(q, carry):
            for b in (0, 1):
                cu = 2 * q + b
                nb = 1 - b

                @pl.when(cu < n_ch2)
                def _():
                    wait_pos_gather(b)

                    @pl.when(cu + 1 < n_ch2)
                    def _():
                        fire_pos_gather(cu + 1, nb)

                    for k in range(8):
                        uix_v[b, pl.ds(k * 16, 16)] = (
                            rawi_v[b, pl.ds(k * 16, 16)] - FROZEN)

                    @pl.when(cu >= 2)
                    def _():
                        pltpu.make_async_copy(
                            urows_v.at[b], out_h.at[upos_v.at[0]],
                            sem_s[b]).wait()

                    pltpu.async_copy(wu_h.at[uix_v.at[b]], urows_v.at[b],
                                     sem_r[b]).wait()
                    pltpu.async_copy(urows_v.at[b], out_h.at[upos_v.at[cu]],
                                     sem_s[b])
            return carry

        lax.fori_loop(0, (n_ch2 + 1) // 2, p2pair, jnp.int32(0))

        @pl.when(n_ch2 >= 1)
        def _():
            pltpu.make_async_copy(urows_v.at[0], out_h.at[upos_v.at[0]],
                                  sem_s[0]).wait()

        @pl.when(n_ch2 >= 2)
        def _():
            pltpu.make_async_copy(urows_v.at[1], out_h.at[upos_v.at[0]],
                                  sem_s[1]).wait()


@jax.jit
def kernel(idx, weight_frozen, weight_unfrozen):
    mesh = plsc.VectorSubcoreMesh(core_axis_name="c", subcore_axis_name="s",
                                  num_cores=NC, num_subcores=NS)
    run = pl.kernel(
        _body,
        out_type=jax.ShapeDtypeStruct((B_ROWS, DIM), jnp.float32),
        mesh=mesh,
        compiler_params=pltpu.CompilerParams(use_tc_tiling_on_sc=False,
                                             needs_layout_passes=False),
        scratch_types=[
            pltpu.VMEM((2, S), jnp.int32),
            pltpu.VMEM((2, NJ, 128), jnp.int32),
            pltpu.VMEM((2, S, DIM), jnp.float32),
            pltpu.VMEM((UROWS, 128), jnp.int32),
            pltpu.VMEM((2, 128), jnp.int32),
            pltpu.VMEM((2, 128), jnp.int32),
            pltpu.VMEM((2, 128, DIM), jnp.float32),
        ] + [pltpu.SemaphoreType.DMA] * 11,
    )
    out = run(idx.reshape(-1), weight_frozen, weight_unfrozen)
    return out.reshape(BATCH, HIST, DIM)


# speedup vs baseline: 2.4944x; 2.4944x over previous
import jax
import jax.numpy as jnp
from jax import lax
from jax.experimental import pallas as pl
from jax.experimental.pallas import tpu as pltpu
from jax.experimental.pallas import tpu_sc as plsc

FROZEN = 900000
UNFROZEN = 100000
DIM = 64
BATCH = 16384
HIST = 50
B_ROWS = BATCH * HIST

NC, NS = 2, 16
NW = NC * NS
PW = B_ROWS // NW
S = 512
NCH = PW // S
G = S // 16
NJ = S // 128
UROWS = (PW + 128 + 127) // 128


def _body(idx_h, wf_h, wu_h, out_h,
          idx_v, fidx_v, rows_v, upos_v, rawi_v, uix_v, urows_v,
          sem_i, sem_g0, sem_g1, sem_w0, sem_w1,
          sem_p0, sem_p1, sem_r0, sem_r1, sem_s0, sem_s1):
    sem_g = (sem_g0, sem_g1)
    sem_w = (sem_w0, sem_w1)
    sem_p = (sem_p0, sem_p1)
    sem_r = (sem_r0, sem_r1)
    sem_s = (sem_s0, sem_s1)
    cid = lax.axis_index("c")
    sid = lax.axis_index("s")
    wid = sid * NC + cid
    wbase = wid * PW
    iota16 = lax.iota(jnp.int32, 16)

    def compact(c, buf, u_off):
        base = wbase + c * S
        for g in range(G):
            v = idx_v[buf, pl.ds(g * 16, 16)]
            mu = v >= FROZEN
            fidx_v[buf, g // 8, pl.ds((g % 8) * 16, 16)] = jnp.where(
                mu, v - FROZEN, v)
            pos = base + g * 16 + iota16
            cs = plsc.cumsum(jnp.where(mu, jnp.int32(1), jnp.int32(0)))
            dst = u_off + cs - 1
            dr = lax.shift_right_logical(dst, 7)
            dc = lax.bitwise_and(dst, 127)
            plsc.store_scatter(upos_v, [dr, dc], pos, mask=mu)
            u_off = u_off + cs[15]
        return u_off

    def fire_gathers(b):
        for j in range(NJ):
            pltpu.async_copy(wf_h.at[fidx_v.at[b, j]],
                             rows_v.at[b, pl.ds(j * 128, 128)], sem_g[b])

    def wait_gathers(b):
        for j in range(NJ):
            pltpu.make_async_copy(wf_h.at[fidx_v.at[b, j]],
                                  rows_v.at[b, pl.ds(j * 128, 128)],
                                  sem_g[b]).wait()

    def fire_write(c, b):
        pltpu.async_copy(rows_v.at[b], out_h.at[pl.ds(wbase + c * S, S)],
                         sem_w[b])

    def wait_write(b):
        pltpu.make_async_copy(rows_v.at[b], out_h.at[pl.ds(0, S)],
                              sem_w[b]).wait()

    def fire_idx_load(c, b):
        pltpu.async_copy(idx_h.at[pl.ds(wbase + c * S, S)], idx_v.at[b], sem_i)

    def wait_idx_load(b):
        pltpu.make_async_copy(idx_h.at[pl.ds(0, S)], idx_v.at[b], sem_i).wait()

    pltpu.sync_copy(idx_h.at[pl.ds(wbase, S)], idx_v.at[0])
    u_off = compact(0, 0, jnp.int32(0))
    fire_gathers(0)
    fire_idx_load(1, 1)

    def pair_body(p, u_off):
        for b in (0, 1):
            c = 2 * p + b
            nb = 1 - b
            wait_idx_load(nb)
            u_off = compact(c + 1, nb, u_off)
            fire_idx_load(c + 2, b)

            @pl.when(c >= 1)
            def _():
                wait_write(nb)

            fire_gathers(nb)
            wait_gathers(b)
            fire_write(c, b)
        return u_off

    u_off = lax.fori_loop(0, (NCH - 2) // 2, pair_body, u_off)

    wait_idx_load(1)
    n_u = compact(NCH - 1, 1, u_off)
    wait_write(1)
    fire_gathers(1)
    wait_gathers(0)
    fire_write(NCH - 2, 0)
    wait_gathers(1)
    fire_write(NCH - 1, 1)
    wait_write(0)
    wait_write(1)

    @pl.when(n_u > 0)
    def _pass2():
        nm1 = n_u - 1
        lr = jnp.full((16,), lax.shift_right_logical(nm1, 7), jnp.int32)
        lc = jnp.full((16,), lax.bitwise_and(nm1, 127), jnp.int32)
        vp = plsc.load_gather(upos_v, [lr, lc])
        for j in range(8):
            dd = n_u + j * 16 + iota16
            ddr = lax.shift_right_logical(dd, 7)
            ddc = lax.bitwise_and(dd, 127)
            plsc.store_scatter(upos_v, [ddr, ddc], vp)
        n_ch2 = (n_u + 127) // 128

        def fire_pos_gather(cu, b):
            pltpu.async_copy(idx_h.at[upos_v.at[cu]], rawi_v.at[b], sem_p[b])

        def wait_pos_gather(b):
            pltpu.make_async_copy(idx_h.at[upos_v.at[0]], rawi_v.at[b],
                                  sem_p[b]).wait()

        fire_pos_gather(0, 0)

        def p2pair(q, carry):
            for b in (0, 1):
                cu = 2 * q + b
                nb = 1 - b

                @pl.when(cu < n_ch2)
                def _():
                    wait_pos_gather(b)

                    @pl.when(cu + 1 < n_ch2)
                    def _():
                        fire_pos_gather(cu + 1, nb)

                    for k in range(8):
                        uix_v[b, pl.ds(k * 16, 16)] = (
                            rawi_v[b, pl.ds(k * 16, 16)] - FROZEN)

                    @pl.when(cu >= 2)
                    def _():
                        pltpu.make_async_copy(
                            urows_v.at[b], out_h.at[upos_v.at[0]],
                            sem_s[b]).wait()

                    pltpu.async_copy(wu_h.at[uix_v.at[b]], urows_v.at[b],
                                     sem_r[b]).wait()
                    pltpu.async_copy(urows_v.at[b], out_h.at[upos_v.at[cu]],
                                     sem_s[b])
            return carry

        lax.fori_loop(0, (n_ch2 + 1) // 2, p2pair, jnp.int32(0))

        @pl.when(n_ch2 >= 1)
        def _():
            pltpu.make_async_copy(urows_v.at[0], out_h.at[upos_v.at[0]],
                                  sem_s[0]).wait()

        @pl.when(n_ch2 >= 2)
        def _():
            pltpu.make_async_copy(urows_v.at[1], out_h.at[upos_v.at[0]],
                                  sem_s[1]).wait()


@jax.jit
def kernel(idx, weight_frozen, weight_unfrozen):
    mesh = plsc.VectorSubcoreMesh(core_axis_name="c", subcore_axis_name="s",
                                  num_cores=NC, num_subcores=NS)
    run = pl.kernel(
        _body,
        out_type=jax.ShapeDtypeStruct((B_ROWS, DIM), jnp.float32),
        mesh=mesh,
        compiler_params=pltpu.CompilerParams(use_tc_tiling_on_sc=False,
                                             needs_layout_passes=False),
        scratch_types=[
            pltpu.VMEM((2, S), jnp.int32),
            pltpu.VMEM((2, NJ, 128), jnp.int32),
            pltpu.VMEM((2, S, DIM), jnp.float32),
            pltpu.VMEM((UROWS, 128), jnp.int32),
            pltpu.VMEM((2, 128), jnp.int32),
            pltpu.VMEM((2, 128), jnp.int32),
            pltpu.VMEM((2, 128, DIM), jnp.float32),
        ] + [pltpu.SemaphoreType.DMA] * 11,
    )
    out = run(idx.reshape(-1), weight_frozen, weight_unfrozen)
    return out.reshape(BATCH, HIST, DIM)
